# baseline (speedup 1.0000x reference)
import functools

import jax
import jax.numpy as jnp
from jax import lax
from jax.experimental import pallas as pl
from jax.experimental.pallas import tpu as pltpu
from jax.experimental.pallas import tpu_sc as plsc

NC = 2
NS = 16
K = 128


def _row_partition(N):
    per = (N // NS) // 8 * 8
    rem = N - per * NS
    return per, rem


def _make_hist(N, E):
    per_tile = E // (NC * NS)
    n_full = per_tile // K
    tail = per_tile - n_full * K
    rpt, rrem = _row_partition(N)
    mesh = plsc.VectorSubcoreMesh(core_axis_name="c", subcore_axis_name="s")

    scratch = [
        pltpu.VMEM((K,), jnp.int32),
        pltpu.VMEM((K, 128), jnp.float32),
        pltpu.VMEM_SHARED((N, 128), jnp.float32),
        pltpu.SemaphoreType.DMA,
    ]
    if tail:
        scratch.insert(1, pltpu.VMEM((tail,), jnp.int32))

    @functools.partial(
        pl.kernel, mesh=mesh,
        out_type=[jax.ShapeDtypeStruct((N, 128), jnp.float32),
                  jax.ShapeDtypeStruct((N, 128), jnp.float32)],
        scratch_types=scratch,
    )
    def hist(col_hbm, zeros_hbm, ones_hbm, c0_hbm, c1_hbm, cidx, *rest):
        if tail:
            cidx_t, ones, acc, sem = rest
        else:
            ones, acc, sem = rest
        c = lax.axis_index("c")
        s = lax.axis_index("s")
        tsl = pl.ds(s * rpt, rpt)
        rsl = pl.ds(N - rrem, rrem)
        pltpu.sync_copy(zeros_hbm.at[tsl], acc.at[tsl])
        if rrem:
            @pl.when(s == NS - 1)
            def _():
                pltpu.sync_copy(zeros_hbm.at[rsl], acc.at[rsl])
        pltpu.sync_copy(ones_hbm, ones)
        plsc.subcore_barrier()

        base0 = (c * NS + s) * per_tile

        def body(j, carry):
            base = base0 + j * K
            pltpu.sync_copy(col_hbm.at[pl.ds(base, K)], cidx)
            pltpu.sync_copy(ones, acc.at[cidx], add=True)
            return carry
        lax.fori_loop(0, n_full, body, 0)
        if tail:
            base = base0 + n_full * K
            pltpu.sync_copy(col_hbm.at[pl.ds(base, tail)], cidx_t)
            pltpu.sync_copy(ones.at[pl.ds(0, tail)], acc.at[cidx_t], add=True)

        plsc.subcore_barrier()

        @pl.when(c == 0)
        def _():
            pltpu.sync_copy(acc.at[tsl], c0_hbm.at[tsl])
            if rrem:
                @pl.when(s == NS - 1)
                def _():
                    pltpu.sync_copy(acc.at[rsl], c0_hbm.at[rsl])

        @pl.when(c == 1)
        def _():
            pltpu.sync_copy(acc.at[tsl], c1_hbm.at[tsl])
            if rrem:
                @pl.when(s == NS - 1)
                def _():
                    pltpu.sync_copy(acc.at[rsl], c1_hbm.at[rsl])

    return hist


def _make_scatter(N, E, HALF):
    per_tile = E // NS
    n_full = per_tile // K
    tail = per_tile - n_full * K
    rpt, rrem = _row_partition(N)
    mesh = plsc.VectorSubcoreMesh(core_axis_name="c", subcore_axis_name="s")

    NBUF = 2
    LOOK = 1
    KS = 128
    NPHASE = 2
    pc = (per_tile // NPHASE) // KS
    IC = pc * KS
    tail = per_tile - NPHASE * IC
    scratch = [
        pltpu.VMEM((IC,), jnp.int32),
        pltpu.VMEM((IC,), jnp.int32),
        [pltpu.VMEM((KS, HALF), jnp.float32) for _ in range(NBUF)],
        pltpu.VMEM_SHARED((N, HALF), jnp.float32),
        [pltpu.SemaphoreType.DMA for _ in range(NBUF)],
        [pltpu.SemaphoreType.DMA for _ in range(NBUF)],
        pltpu.SemaphoreType.DMA,
    ]
    if tail:
        scratch.insert(2, pltpu.VMEM((tail,), jnp.int32))
        scratch.insert(3, pltpu.VMEM((tail,), jnp.int32))
        scratch.insert(4, pltpu.VMEM((tail, HALF), jnp.float32))

    @functools.partial(
        pl.kernel, mesh=mesh,
        out_type=[jax.ShapeDtypeStruct((N, HALF), jnp.float32),
                  jax.ShapeDtypeStruct((N, HALF), jnp.float32)],
        scratch_types=scratch,
    )
    def scatter(row_hbm, col_hbm, xs0_hbm, xs1_hbm, zeros_hbm,
                s0_hbm, s1_hbm, ridx, cidx, *rest):
        if tail:
            ridx_t, cidx_t, rows_t, rows, acc, gsem, ssem, isem = rest
        else:
            rows, acc, gsem, ssem, isem = rest
        c = lax.axis_index("c")
        s = lax.axis_index("s")
        tsl = pl.ds(s * rpt, rpt)
        rsl = pl.ds(N - rrem, rrem)
        base0 = s * per_tile
        ld_r = pltpu.async_copy(row_hbm.at[pl.ds(base0, IC)], ridx, isem)
        ld_c = pltpu.async_copy(col_hbm.at[pl.ds(base0, IC)], cidx, isem)
        pltpu.sync_copy(zeros_hbm.at[tsl], acc.at[tsl])
        if rrem:
            @pl.when(s == NS - 1)
            def _():
                pltpu.sync_copy(zeros_hbm.at[rsl], acc.at[rsl])
        ld_r.wait()
        ld_c.wait()
        plsc.subcore_barrier()

        n_groups = (pc + NBUF - 1) // NBUF

        def gwait(xs_hbm, b):
            pltpu.make_async_copy(
                xs_hbm.at[ridx.at[pl.ds(0, KS)]], rows[b], gsem[b]).wait()

        def swait(b):
            pltpu.make_async_copy(
                rows[b], acc.at[cidx.at[pl.ds(0, KS)]], ssem[b]).wait()

        def do_edges(xs_hbm):
            for p in range(NPHASE):
                if p > 0:
                    pltpu.sync_copy(
                        row_hbm.at[pl.ds(base0 + p * IC, IC)], ridx)
                    pltpu.sync_copy(
                        col_hbm.at[pl.ds(base0 + p * IC, IC)], cidx)
                for b in range(min(LOOK, pc)):
                    pltpu.async_copy(
                        xs_hbm.at[ridx.at[pl.ds(b * KS, KS)]], rows[b],
                        gsem[b])

                def group(g, carry):
                    for b in range(NBUF):
                        j = g * NBUF + b

                        @pl.when(j < pc)
                        def _():
                            gwait(xs_hbm, b)
                            pltpu.async_copy(
                                rows[b], acc.at[cidx.at[pl.ds(j * KS, KS)]],
                                ssem[b], add=True)
                            b2 = (b + LOOK) % NBUF
                            nxt = j + LOOK

                            @pl.when((nxt < pc) & (j >= LOOK))
                            def _():
                                swait(b2)

                            @pl.when(nxt < pc)
                            def _():
                                pltpu.async_copy(
                                    xs_hbm.at[ridx.at[pl.ds(nxt * KS, KS)]],
                                    rows[b2], gsem[b2])
                    return carry
                lax.fori_loop(0, n_groups, group, 0)
                for last in range(max(pc - NBUF, 0), pc):
                    swait(last % NBUF)
            if tail:
                base = base0 + NPHASE * IC
                pltpu.sync_copy(row_hbm.at[pl.ds(base, tail)], ridx_t)
                pltpu.sync_copy(col_hbm.at[pl.ds(base, tail)], cidx_t)
                pltpu.async_copy(xs_hbm.at[ridx_t], rows_t, isem).wait()
                pltpu.sync_copy(rows_t, acc.at[cidx_t], add=True)

        @pl.when(c == 0)
        def _():
            do_edges(xs0_hbm)

        @pl.when(c == 1)
        def _():
            do_edges(xs1_hbm)

        plsc.subcore_barrier()

        @pl.when(c == 0)
        def _():
            pltpu.sync_copy(acc.at[tsl], s0_hbm.at[tsl])
            if rrem:
                @pl.when(s == NS - 1)
                def _():
                    pltpu.sync_copy(acc.at[rsl], s0_hbm.at[rsl])

        @pl.when(c == 1)
        def _():
            pltpu.sync_copy(acc.at[tsl], s1_hbm.at[tsl])
            if rrem:
                @pl.when(s == NS - 1)
                def _():
                    pltpu.sync_copy(acc.at[rsl], s1_hbm.at[rsl])

    return scatter


_DN = (((1,), (1,)), ((), ()))


def _dis_from_counts(c0, c1):
    deg = c0[:, 0:1] + c1[:, 0:1] + 1.0
    return lax.rsqrt(deg)


def _tc1_body(x_ref, W_ref, Wres_ref, bres_ref, c0_ref, c1_ref,
              xs0_ref, xs1_ref, g_ref):
    xb = x_ref[...]
    dis = _dis_from_counts(c0_ref[...], c1_ref[...])
    xl = lax.dot_general(xb, W_ref[...], _DN,
                         preferred_element_type=jnp.float32)
    xs = xl * dis
    h = xs.shape[1] // 2
    xs0_ref[...] = xs[:, :h]
    xs1_ref[...] = xs[:, h:]
    gl = lax.dot_general(xb, Wres_ref[...], _DN,
                         preferred_element_type=jnp.float32)
    g_ref[...] = jax.nn.sigmoid(gl + bres_ref[...])


def _tc2_body(x_ref, g_ref, S0_ref, S1_ref, xs0_ref, xs1_ref, b_ref,
              c0_ref, c1_ref, W_ref, Wres_ref, bres_ref,
              h_ref, ys0_ref, ys1_ref, g2_ref):
    xb = x_ref[...]
    dis = _dis_from_counts(c0_ref[...], c1_ref[...])
    S = jnp.concatenate([S0_ref[...], S1_ref[...]], axis=1)
    xs = jnp.concatenate([xs0_ref[...], xs1_ref[...]], axis=1)
    out = dis * (S + xs) + b_ref[...]
    g = g_ref[...]
    h1 = jnp.maximum((1.0 - g) * xb + g * out, 0.0)
    h_ref[...] = h1
    xl2 = lax.dot_general(h1, W_ref[...], _DN,
                          preferred_element_type=jnp.float32)
    ys = xl2 * dis
    h = ys.shape[1] // 2
    ys0_ref[...] = ys[:, :h]
    ys1_ref[...] = ys[:, h:]
    gl2 = lax.dot_general(h1, Wres_ref[...], _DN,
                          preferred_element_type=jnp.float32)
    g2_ref[...] = jax.nn.sigmoid(gl2 + bres_ref[...])


def _tc3_body(x_ref, g_ref, S0_ref, S1_ref, xs0_ref, xs1_ref, b_ref,
              c0_ref, c1_ref, out_ref):
    xb = x_ref[...]
    dis = _dis_from_counts(c0_ref[...], c1_ref[...])
    S = jnp.concatenate([S0_ref[...], S1_ref[...]], axis=1)
    xs = jnp.concatenate([xs0_ref[...], xs1_ref[...]], axis=1)
    out = dis * (S + xs) + b_ref[...]
    g = g_ref[...]
    out_ref[...] = jnp.maximum((1.0 - g) * xb + g * out, 0.0)


def kernel(x, edge_index, W1, b1, Wres1, bres1, W2, b2, Wres2, bres2):
    N, D = x.shape
    E = edge_index.shape[1]
    HALF = D // 2
    BN = 1000
    grid = (N // BN,)

    row = edge_index[0]
    col = edge_index[1]
    zeros_half = jnp.zeros((N, HALF), jnp.float32)
    zeros_cnt = zeros_half
    b1r = b1.reshape(1, D)
    b2r = b2.reshape(1, D)
    bres1r = bres1.reshape(1, D)
    bres2r = bres2.reshape(1, D)

    ones_cnt = jnp.ones((K, 128), jnp.float32)
    c0, c1 = _make_hist(N, E)(col, zeros_cnt, ones_cnt)

    row_spec = pl.BlockSpec((BN, D), lambda i: (i, 0))
    half_spec = pl.BlockSpec((BN, HALF), lambda i: (i, 0))
    cnt_spec = pl.BlockSpec((BN, 128), lambda i: (i, 0))
    w_spec = pl.BlockSpec((D, D), lambda i: (0, 0))
    b_spec = pl.BlockSpec((1, D), lambda i: (0, 0))

    xs0, xs1, g1 = pl.pallas_call(
        _tc1_body,
        grid=grid,
        in_specs=[row_spec, w_spec, w_spec, b_spec, cnt_spec, cnt_spec],
        out_specs=[half_spec, half_spec, row_spec],
        out_shape=[jax.ShapeDtypeStruct((N, HALF), jnp.float32),
                   jax.ShapeDtypeStruct((N, HALF), jnp.float32),
                   jax.ShapeDtypeStruct((N, D), jnp.float32)],
    )(x, W1, Wres1, bres1r, c0, c1)

    scatter = _make_scatter(N, E, HALF)
    S0, S1 = scatter(row, col, xs0, xs1, zeros_half)

    h1, ys0, ys1, g2 = pl.pallas_call(
        _tc2_body,
        grid=grid,
        in_specs=[row_spec, row_spec, half_spec, half_spec, half_spec,
                  half_spec, b_spec, cnt_spec, cnt_spec, w_spec, w_spec,
                  b_spec],
        out_specs=[row_spec, half_spec, half_spec, row_spec],
        out_shape=[jax.ShapeDtypeStruct((N, D), jnp.float32),
                   jax.ShapeDtypeStruct((N, HALF), jnp.float32),
                   jax.ShapeDtypeStruct((N, HALF), jnp.float32),
                   jax.ShapeDtypeStruct((N, D), jnp.float32)],
    )(x, g1, S0, S1, xs0, xs1, b1r, c0, c1, W2, Wres2, bres2r)

    T0, T1 = scatter(row, col, ys0, ys1, zeros_half)

    out = pl.pallas_call(
        _tc3_body,
        grid=grid,
        in_specs=[row_spec, row_spec, half_spec, half_spec, half_spec,
                  half_spec, b_spec, cnt_spec, cnt_spec],
        out_specs=row_spec,
        out_shape=jax.ShapeDtypeStruct((N, D), jnp.float32),
    )(h1, g2, T0, T1, ys0, ys1, b2r, c0, c1)

    return out


# speedup vs baseline: 1.1089x; 1.1089x over previous
import functools

import jax
import jax.numpy as jnp
from jax import lax
from jax.experimental import pallas as pl
from jax.experimental.pallas import tpu as pltpu
from jax.experimental.pallas import tpu_sc as plsc

NC = 2
NS = 16
K = 128


def _row_partition(N):
    per = (N // NS) // 8 * 8
    rem = N - per * NS
    return per, rem


def _make_hist(N, E):
    per_tile = E // (NC * NS)
    n_full = per_tile // K
    tail = per_tile - n_full * K
    rpt, rrem = _row_partition(N)
    mesh = plsc.VectorSubcoreMesh(core_axis_name="c", subcore_axis_name="s")

    scratch = [
        pltpu.VMEM((K,), jnp.int32),
        pltpu.VMEM((K, 128), jnp.float32),
        pltpu.VMEM_SHARED((N, 128), jnp.float32),
        pltpu.SemaphoreType.DMA,
    ]
    if tail:
        scratch.insert(1, pltpu.VMEM((tail,), jnp.int32))

    @functools.partial(
        pl.kernel, mesh=mesh,
        out_type=[jax.ShapeDtypeStruct((N, 128), jnp.float32),
                  jax.ShapeDtypeStruct((N, 128), jnp.float32)],
        scratch_types=scratch,
    )
    def hist(col_hbm, zeros_hbm, ones_hbm, c0_hbm, c1_hbm, cidx, *rest):
        if tail:
            cidx_t, ones, acc, sem = rest
        else:
            ones, acc, sem = rest
        c = lax.axis_index("c")
        s = lax.axis_index("s")
        tsl = pl.ds(s * rpt, rpt)
        rsl = pl.ds(N - rrem, rrem)
        pltpu.sync_copy(zeros_hbm.at[tsl], acc.at[tsl])
        if rrem:
            @pl.when(s == NS - 1)
            def _():
                pltpu.sync_copy(zeros_hbm.at[rsl], acc.at[rsl])
        pltpu.sync_copy(ones_hbm, ones)
        plsc.subcore_barrier()

        base0 = (c * NS + s) * per_tile

        def body(j, carry):
            base = base0 + j * K
            pltpu.sync_copy(col_hbm.at[pl.ds(base, K)], cidx)
            pltpu.sync_copy(ones, acc.at[cidx], add=True)
            return carry
        lax.fori_loop(0, n_full, body, 0)
        if tail:
            base = base0 + n_full * K
            pltpu.sync_copy(col_hbm.at[pl.ds(base, tail)], cidx_t)
            pltpu.sync_copy(ones.at[pl.ds(0, tail)], acc.at[cidx_t], add=True)

        plsc.subcore_barrier()

        @pl.when(c == 0)
        def _():
            pltpu.sync_copy(acc.at[tsl], c0_hbm.at[tsl])
            if rrem:
                @pl.when(s == NS - 1)
                def _():
                    pltpu.sync_copy(acc.at[rsl], c0_hbm.at[rsl])

        @pl.when(c == 1)
        def _():
            pltpu.sync_copy(acc.at[tsl], c1_hbm.at[tsl])
            if rrem:
                @pl.when(s == NS - 1)
                def _():
                    pltpu.sync_copy(acc.at[rsl], c1_hbm.at[rsl])

    return hist


def _make_scatter(N, E, HALF):
    per_tile = E // NS
    n_full = per_tile // K
    tail = per_tile - n_full * K
    rpt, rrem = _row_partition(N)
    mesh = plsc.VectorSubcoreMesh(core_axis_name="c", subcore_axis_name="s")

    NBUF = 2
    LOOK = 1
    KS = 128
    NPHASE = 2
    pc = (per_tile // NPHASE) // KS
    IC = pc * KS
    tail = per_tile - NPHASE * IC
    scratch = [
        pltpu.VMEM((IC,), jnp.int32),
        pltpu.VMEM((IC,), jnp.int32),
        [pltpu.VMEM((KS, HALF), jnp.float32) for _ in range(NBUF)],
        pltpu.VMEM_SHARED((N, HALF), jnp.float32),
        [pltpu.SemaphoreType.DMA for _ in range(NBUF)],
        pltpu.SemaphoreType.DMA,
    ]
    if tail:
        scratch.insert(2, pltpu.VMEM((tail,), jnp.int32))
        scratch.insert(3, pltpu.VMEM((tail,), jnp.int32))
        scratch.insert(4, pltpu.VMEM((tail, HALF), jnp.float32))

    @functools.partial(
        pl.kernel, mesh=mesh,
        out_type=[jax.ShapeDtypeStruct((N, HALF), jnp.float32),
                  jax.ShapeDtypeStruct((N, HALF), jnp.float32)],
        scratch_types=scratch,
    )
    def scatter(row_hbm, col_hbm, xs0_hbm, xs1_hbm, zeros_hbm,
                s0_hbm, s1_hbm, ridx, cidx, *rest):
        if tail:
            ridx_t, cidx_t, rows_t, rows, acc, gsem, isem = rest
        else:
            rows, acc, gsem, isem = rest
        c = lax.axis_index("c")
        s = lax.axis_index("s")
        tsl = pl.ds(s * rpt, rpt)
        rsl = pl.ds(N - rrem, rrem)
        base0 = s * per_tile
        ld_r = pltpu.async_copy(row_hbm.at[pl.ds(base0, IC)], ridx, isem)
        ld_c = pltpu.async_copy(col_hbm.at[pl.ds(base0, IC)], cidx, isem)
        pltpu.sync_copy(zeros_hbm.at[tsl], acc.at[tsl])
        if rrem:
            @pl.when(s == NS - 1)
            def _():
                pltpu.sync_copy(zeros_hbm.at[rsl], acc.at[rsl])
        ld_r.wait()
        ld_c.wait()
        plsc.subcore_barrier()

        n_groups = (pc + NBUF - 1) // NBUF

        def gwait(xs_hbm, b):
            pltpu.make_async_copy(
                xs_hbm.at[ridx.at[pl.ds(0, KS)]], rows[b], gsem[b]).wait()

        def do_edges(xs_hbm):
            for p in range(NPHASE):
                if p > 0:
                    pltpu.sync_copy(
                        row_hbm.at[pl.ds(base0 + p * IC, IC)], ridx)
                    pltpu.sync_copy(
                        col_hbm.at[pl.ds(base0 + p * IC, IC)], cidx)
                for b in range(min(NBUF, pc)):
                    pltpu.async_copy(
                        xs_hbm.at[ridx.at[pl.ds(b * KS, KS)]], rows[b],
                        gsem[b])

                def group(g, carry):
                    for b in range(NBUF):
                        j = g * NBUF + b

                        @pl.when(j < pc)
                        def _():
                            gwait(xs_hbm, b)
                            pltpu.sync_copy(
                                rows[b], acc.at[cidx.at[pl.ds(j * KS, KS)]],
                                add=True)
                            nxt = j + NBUF

                            @pl.when(nxt < pc)
                            def _():
                                pltpu.async_copy(
                                    xs_hbm.at[ridx.at[pl.ds(nxt * KS, KS)]],
                                    rows[b], gsem[b])
                    return carry
                lax.fori_loop(0, n_groups, group, 0)
            if tail:
                base = base0 + NPHASE * IC
                pltpu.sync_copy(row_hbm.at[pl.ds(base, tail)], ridx_t)
                pltpu.sync_copy(col_hbm.at[pl.ds(base, tail)], cidx_t)
                pltpu.async_copy(xs_hbm.at[ridx_t], rows_t, isem).wait()
                pltpu.sync_copy(rows_t, acc.at[cidx_t], add=True)

        @pl.when(c == 0)
        def _():
            do_edges(xs0_hbm)

        @pl.when(c == 1)
        def _():
            do_edges(xs1_hbm)

        plsc.subcore_barrier()

        @pl.when(c == 0)
        def _():
            pltpu.sync_copy(acc.at[tsl], s0_hbm.at[tsl])
            if rrem:
                @pl.when(s == NS - 1)
                def _():
                    pltpu.sync_copy(acc.at[rsl], s0_hbm.at[rsl])

        @pl.when(c == 1)
        def _():
            pltpu.sync_copy(acc.at[tsl], s1_hbm.at[tsl])
            if rrem:
                @pl.when(s == NS - 1)
                def _():
                    pltpu.sync_copy(acc.at[rsl], s1_hbm.at[rsl])

    return scatter


_DN = (((1,), (1,)), ((), ()))


def _dis_from_counts(c0, c1):
    deg = c0[:, 0:1] + c1[:, 0:1] + 1.0
    return lax.rsqrt(deg)


def _tc1a_body(x_ref, W_ref, Wres_ref, bres_ref, xl_ref, g_ref):
    xb = x_ref[...]
    xl_ref[...] = lax.dot_general(xb, W_ref[...], _DN,
                                  preferred_element_type=jnp.float32)
    gl = lax.dot_general(xb, Wres_ref[...], _DN,
                         preferred_element_type=jnp.float32)
    g_ref[...] = jax.nn.sigmoid(gl + bres_ref[...])


def _tc1b_body(xl_ref, c0_ref, c1_ref, xs0_ref, xs1_ref):
    dis = _dis_from_counts(c0_ref[...], c1_ref[...])
    xs = xl_ref[...] * dis
    h = xs.shape[1] // 2
    xs0_ref[...] = xs[:, :h]
    xs1_ref[...] = xs[:, h:]


def _tc2_body(x_ref, g_ref, S0_ref, S1_ref, xs0_ref, xs1_ref, b_ref,
              c0_ref, c1_ref, W_ref, Wres_ref, bres_ref,
              h_ref, ys0_ref, ys1_ref, g2_ref):
    xb = x_ref[...]
    dis = _dis_from_counts(c0_ref[...], c1_ref[...])
    S = jnp.concatenate([S0_ref[...], S1_ref[...]], axis=1)
    xs = jnp.concatenate([xs0_ref[...], xs1_ref[...]], axis=1)
    out = dis * (S + xs) + b_ref[...]
    g = g_ref[...]
    h1 = jnp.maximum((1.0 - g) * xb + g * out, 0.0)
    h_ref[...] = h1
    xl2 = lax.dot_general(h1, W_ref[...], _DN,
                          preferred_element_type=jnp.float32)
    ys = xl2 * dis
    h = ys.shape[1] // 2
    ys0_ref[...] = ys[:, :h]
    ys1_ref[...] = ys[:, h:]
    gl2 = lax.dot_general(h1, Wres_ref[...], _DN,
                          preferred_element_type=jnp.float32)
    g2_ref[...] = jax.nn.sigmoid(gl2 + bres_ref[...])


def _tc3_body(x_ref, g_ref, S0_ref, S1_ref, xs0_ref, xs1_ref, b_ref,
              c0_ref, c1_ref, out_ref):
    xb = x_ref[...]
    dis = _dis_from_counts(c0_ref[...], c1_ref[...])
    S = jnp.concatenate([S0_ref[...], S1_ref[...]], axis=1)
    xs = jnp.concatenate([xs0_ref[...], xs1_ref[...]], axis=1)
    out = dis * (S + xs) + b_ref[...]
    g = g_ref[...]
    out_ref[...] = jnp.maximum((1.0 - g) * xb + g * out, 0.0)


def kernel(x, edge_index, W1, b1, Wres1, bres1, W2, b2, Wres2, bres2):
    N, D = x.shape
    E = edge_index.shape[1]
    HALF = D // 2
    BN = 1000
    grid = (N // BN,)

    row = edge_index[0]
    col = edge_index[1]
    zeros_half = jnp.zeros((N, HALF), jnp.float32)
    zeros_cnt = zeros_half
    b1r = b1.reshape(1, D)
    b2r = b2.reshape(1, D)
    bres1r = bres1.reshape(1, D)
    bres2r = bres2.reshape(1, D)

    ones_cnt = jnp.ones((K, 128), jnp.float32)
    c0, c1 = _make_hist(N, E)(col, zeros_cnt, ones_cnt)

    row_spec = pl.BlockSpec((BN, D), lambda i: (i, 0))
    half_spec = pl.BlockSpec((BN, HALF), lambda i: (i, 0))
    cnt_spec = pl.BlockSpec((BN, 128), lambda i: (i, 0))
    w_spec = pl.BlockSpec((D, D), lambda i: (0, 0))
    b_spec = pl.BlockSpec((1, D), lambda i: (0, 0))

    xl1, g1 = pl.pallas_call(
        _tc1a_body,
        grid=grid,
        in_specs=[row_spec, w_spec, w_spec, b_spec],
        out_specs=[row_spec, row_spec],
        out_shape=[jax.ShapeDtypeStruct((N, D), jnp.float32),
                   jax.ShapeDtypeStruct((N, D), jnp.float32)],
    )(x, W1, Wres1, bres1r)

    xs0, xs1 = pl.pallas_call(
        _tc1b_body,
        grid=grid,
        in_specs=[row_spec, cnt_spec, cnt_spec],
        out_specs=[half_spec, half_spec],
        out_shape=[jax.ShapeDtypeStruct((N, HALF), jnp.float32),
                   jax.ShapeDtypeStruct((N, HALF), jnp.float32)],
    )(xl1, c0, c1)

    scatter = _make_scatter(N, E, HALF)
    S0, S1 = scatter(row, col, xs0, xs1, zeros_half)

    h1, ys0, ys1, g2 = pl.pallas_call(
        _tc2_body,
        grid=grid,
        in_specs=[row_spec, row_spec, half_spec, half_spec, half_spec,
                  half_spec, b_spec, cnt_spec, cnt_spec, w_spec, w_spec,
                  b_spec],
        out_specs=[row_spec, half_spec, half_spec, row_spec],
        out_shape=[jax.ShapeDtypeStruct((N, D), jnp.float32),
                   jax.ShapeDtypeStruct((N, HALF), jnp.float32),
                   jax.ShapeDtypeStruct((N, HALF), jnp.float32),
                   jax.ShapeDtypeStruct((N, D), jnp.float32)],
    )(x, g1, S0, S1, xs0, xs1, b1r, c0, c1, W2, Wres2, bres2r)

    T0, T1 = scatter(row, col, ys0, ys1, zeros_half)

    out = pl.pallas_call(
        _tc3_body,
        grid=grid,
        in_specs=[row_spec, row_spec, half_spec, half_spec, half_spec,
                  half_spec, b_spec, cnt_spec, cnt_spec],
        out_specs=row_spec,
        out_shape=jax.ShapeDtypeStruct((N, D), jnp.float32),
    )(h1, g2, T0, T1, ys0, ys1, b2r, c0, c1)

    return out


# speedup vs baseline: 1.1745x; 1.0591x over previous
import functools

import jax
import jax.numpy as jnp
from jax import lax
from jax.experimental import pallas as pl
from jax.experimental.pallas import tpu as pltpu
from jax.experimental.pallas import tpu_sc as plsc

NC = 2
NS = 16
K = 128


def _row_partition(N):
    per = (N // NS) // 8 * 8
    rem = N - per * NS
    return per, rem


def _make_hist(N, E):
    per_tile = E // (NC * NS)
    n_full = per_tile // K
    tail = per_tile - n_full * K
    rpt, rrem = _row_partition(N)
    WIN = 16
    mesh = plsc.VectorSubcoreMesh(core_axis_name="c", subcore_axis_name="s")

    scratch = [
        pltpu.VMEM((per_tile,), jnp.int32),
        pltpu.VMEM((K, 128), jnp.float32),
        pltpu.VMEM_SHARED((N, 128), jnp.float32),
        pltpu.SemaphoreType.DMA,
        pltpu.SemaphoreType.DMA,
    ]

    @functools.partial(
        pl.kernel, mesh=mesh,
        out_type=[jax.ShapeDtypeStruct((N, 128), jnp.float32),
                  jax.ShapeDtypeStruct((N, 128), jnp.float32)],
        scratch_types=scratch,
    )
    def hist(col_hbm, zeros_hbm, ones_hbm, c0_hbm, c1_hbm,
             cidx, ones, acc, sem, isem):
        c = lax.axis_index("c")
        s = lax.axis_index("s")
        tsl = pl.ds(s * rpt, rpt)
        rsl = pl.ds(N - rrem, rrem)
        base0 = (c * NS + s) * per_tile
        ld = pltpu.async_copy(col_hbm.at[pl.ds(base0, per_tile)], cidx, isem)
        pltpu.sync_copy(zeros_hbm.at[tsl], acc.at[tsl])
        if rrem:
            @pl.when(s == NS - 1)
            def _():
                pltpu.sync_copy(zeros_hbm.at[rsl], acc.at[rsl])
        pltpu.sync_copy(ones_hbm, ones)
        ld.wait()
        plsc.subcore_barrier()

        def drain_one():
            pltpu.make_async_copy(
                ones, acc.at[cidx.at[pl.ds(0, K)]], sem).wait()

        def body(j, carry):
            pltpu.async_copy(
                ones, acc.at[cidx.at[pl.ds(j * K, K)]], sem, add=True)

            @pl.when(j >= WIN)
            def _():
                drain_one()
            return carry
        lax.fori_loop(0, n_full, body, 0)

        def drain(j, carry):
            drain_one()
            return carry
        lax.fori_loop(0, min(WIN, n_full), drain, 0)
        if tail:
            pltpu.async_copy(
                ones.at[pl.ds(0, tail)],
                acc.at[cidx.at[pl.ds(n_full * K, tail)]], isem,
                add=True).wait()

        plsc.subcore_barrier()

        @pl.when(c == 0)
        def _():
            pltpu.sync_copy(acc.at[tsl], c0_hbm.at[tsl])
            if rrem:
                @pl.when(s == NS - 1)
                def _():
                    pltpu.sync_copy(acc.at[rsl], c0_hbm.at[rsl])

        @pl.when(c == 1)
        def _():
            pltpu.sync_copy(acc.at[tsl], c1_hbm.at[tsl])
            if rrem:
                @pl.when(s == NS - 1)
                def _():
                    pltpu.sync_copy(acc.at[rsl], c1_hbm.at[rsl])

    return hist


def _make_scatter(N, E, HALF):
    per_tile = E // NS
    n_full = per_tile // K
    tail = per_tile - n_full * K
    rpt, rrem = _row_partition(N)
    mesh = plsc.VectorSubcoreMesh(core_axis_name="c", subcore_axis_name="s")

    NBUF = 2
    LOOK = 1
    KS = 128
    NPHASE = 2
    pc = (per_tile // NPHASE) // KS
    IC = pc * KS
    tail = per_tile - NPHASE * IC
    scratch = [
        pltpu.VMEM((IC,), jnp.int32),
        pltpu.VMEM((IC,), jnp.int32),
        [pltpu.VMEM((KS, HALF), jnp.float32) for _ in range(NBUF)],
        pltpu.VMEM_SHARED((N, HALF), jnp.float32),
        [pltpu.SemaphoreType.DMA for _ in range(NBUF)],
        pltpu.SemaphoreType.DMA,
    ]
    if tail:
        scratch.insert(2, pltpu.VMEM((tail,), jnp.int32))
        scratch.insert(3, pltpu.VMEM((tail,), jnp.int32))
        scratch.insert(4, pltpu.VMEM((tail, HALF), jnp.float32))

    @functools.partial(
        pl.kernel, mesh=mesh,
        out_type=[jax.ShapeDtypeStruct((N, HALF), jnp.float32),
                  jax.ShapeDtypeStruct((N, HALF), jnp.float32)],
        scratch_types=scratch,
    )
    def scatter(row_hbm, col_hbm, xs0_hbm, xs1_hbm, zeros_hbm,
                s0_hbm, s1_hbm, ridx, cidx, *rest):
        if tail:
            ridx_t, cidx_t, rows_t, rows, acc, gsem, isem = rest
        else:
            rows, acc, gsem, isem = rest
        c = lax.axis_index("c")
        s = lax.axis_index("s")
        tsl = pl.ds(s * rpt, rpt)
        rsl = pl.ds(N - rrem, rrem)
        base0 = s * per_tile
        ld_r = pltpu.async_copy(row_hbm.at[pl.ds(base0, IC)], ridx, isem)
        ld_c = pltpu.async_copy(col_hbm.at[pl.ds(base0, IC)], cidx, isem)
        pltpu.sync_copy(zeros_hbm.at[tsl], acc.at[tsl])
        if rrem:
            @pl.when(s == NS - 1)
            def _():
                pltpu.sync_copy(zeros_hbm.at[rsl], acc.at[rsl])
        ld_r.wait()
        ld_c.wait()
        plsc.subcore_barrier()

        n_groups = (pc + NBUF - 1) // NBUF

        def gwait(xs_hbm, b):
            pltpu.make_async_copy(
                xs_hbm.at[ridx.at[pl.ds(0, KS)]], rows[b], gsem[b]).wait()

        def do_edges(xs_hbm):
            for p in range(NPHASE):
                if p > 0:
                    pltpu.sync_copy(
                        row_hbm.at[pl.ds(base0 + p * IC, IC)], ridx)
                    pltpu.sync_copy(
                        col_hbm.at[pl.ds(base0 + p * IC, IC)], cidx)
                for b in range(min(NBUF, pc)):
                    pltpu.async_copy(
                        xs_hbm.at[ridx.at[pl.ds(b * KS, KS)]], rows[b],
                        gsem[b])

                def group(g, carry):
                    for b in range(NBUF):
                        j = g * NBUF + b

                        @pl.when(j < pc)
                        def _():
                            gwait(xs_hbm, b)
                            pltpu.sync_copy(
                                rows[b], acc.at[cidx.at[pl.ds(j * KS, KS)]],
                                add=True)
                            nxt = j + NBUF

                            @pl.when(nxt < pc)
                            def _():
                                pltpu.async_copy(
                                    xs_hbm.at[ridx.at[pl.ds(nxt * KS, KS)]],
                                    rows[b], gsem[b])
                    return carry
                lax.fori_loop(0, n_groups, group, 0)
            if tail:
                base = base0 + NPHASE * IC
                pltpu.sync_copy(row_hbm.at[pl.ds(base, tail)], ridx_t)
                pltpu.sync_copy(col_hbm.at[pl.ds(base, tail)], cidx_t)
                pltpu.async_copy(xs_hbm.at[ridx_t], rows_t, isem).wait()
                pltpu.sync_copy(rows_t, acc.at[cidx_t], add=True)

        @pl.when(c == 0)
        def _():
            do_edges(xs0_hbm)

        @pl.when(c == 1)
        def _():
            do_edges(xs1_hbm)

        plsc.subcore_barrier()

        @pl.when(c == 0)
        def _():
            pltpu.sync_copy(acc.at[tsl], s0_hbm.at[tsl])
            if rrem:
                @pl.when(s == NS - 1)
                def _():
                    pltpu.sync_copy(acc.at[rsl], s0_hbm.at[rsl])

        @pl.when(c == 1)
        def _():
            pltpu.sync_copy(acc.at[tsl], s1_hbm.at[tsl])
            if rrem:
                @pl.when(s == NS - 1)
                def _():
                    pltpu.sync_copy(acc.at[rsl], s1_hbm.at[rsl])

    return scatter


_DN = (((1,), (1,)), ((), ()))


def _dis_from_counts(c0, c1):
    deg = c0[:, 0:1] + c1[:, 0:1] + 1.0
    return lax.rsqrt(deg)


def _mm_body(x_ref, W_ref, xl_ref):
    xl_ref[...] = lax.dot_general(x_ref[...], W_ref[...], _DN,
                                  preferred_element_type=jnp.float32)


def _gate_body(x_ref, Wres_ref, bres_ref, g_ref):
    gl = lax.dot_general(x_ref[...], Wres_ref[...], _DN,
                         preferred_element_type=jnp.float32)
    g_ref[...] = jax.nn.sigmoid(gl + bres_ref[...])


def _tc1b_body(xl_ref, c0_ref, c1_ref, xs0_ref, xs1_ref):
    dis = _dis_from_counts(c0_ref[...], c1_ref[...])
    xs = xl_ref[...] * dis
    h = xs.shape[1] // 2
    xs0_ref[...] = xs[:, :h]
    xs1_ref[...] = xs[:, h:]


def _tc2a_body(x_ref, g_ref, S0_ref, S1_ref, xs0_ref, xs1_ref, b_ref,
               c0_ref, c1_ref, W_ref, h_ref, ys0_ref, ys1_ref):
    xb = x_ref[...]
    dis = _dis_from_counts(c0_ref[...], c1_ref[...])
    S = jnp.concatenate([S0_ref[...], S1_ref[...]], axis=1)
    xs = jnp.concatenate([xs0_ref[...], xs1_ref[...]], axis=1)
    out = dis * (S + xs) + b_ref[...]
    g = g_ref[...]
    h1 = jnp.maximum((1.0 - g) * xb + g * out, 0.0)
    h_ref[...] = h1
    xl2 = lax.dot_general(h1, W_ref[...], _DN,
                          preferred_element_type=jnp.float32)
    ys = xl2 * dis
    h = ys.shape[1] // 2
    ys0_ref[...] = ys[:, :h]
    ys1_ref[...] = ys[:, h:]


def _tc3_body(x_ref, g_ref, S0_ref, S1_ref, xs0_ref, xs1_ref, b_ref,
              c0_ref, c1_ref, out_ref):
    xb = x_ref[...]
    dis = _dis_from_counts(c0_ref[...], c1_ref[...])
    S = jnp.concatenate([S0_ref[...], S1_ref[...]], axis=1)
    xs = jnp.concatenate([xs0_ref[...], xs1_ref[...]], axis=1)
    out = dis * (S + xs) + b_ref[...]
    g = g_ref[...]
    out_ref[...] = jnp.maximum((1.0 - g) * xb + g * out, 0.0)


def kernel(x, edge_index, W1, b1, Wres1, bres1, W2, b2, Wres2, bres2):
    N, D = x.shape
    E = edge_index.shape[1]
    HALF = D // 2
    BN = 1000
    grid = (N // BN,)

    row = edge_index[0]
    col = edge_index[1]
    zeros_half = jnp.zeros((N, HALF), jnp.float32)
    zeros_cnt = zeros_half
    b1r = b1.reshape(1, D)
    b2r = b2.reshape(1, D)
    bres1r = bres1.reshape(1, D)
    bres2r = bres2.reshape(1, D)

    ones_cnt = jnp.ones((K, 128), jnp.float32)
    c0, c1 = _make_hist(N, E)(col, zeros_cnt, ones_cnt)

    row_spec = pl.BlockSpec((BN, D), lambda i: (i, 0))
    half_spec = pl.BlockSpec((BN, HALF), lambda i: (i, 0))
    cnt_spec = pl.BlockSpec((BN, 128), lambda i: (i, 0))
    w_spec = pl.BlockSpec((D, D), lambda i: (0, 0))
    b_spec = pl.BlockSpec((1, D), lambda i: (0, 0))

    mm = pl.pallas_call(
        _mm_body,
        grid=grid,
        in_specs=[row_spec, w_spec],
        out_specs=row_spec,
        out_shape=jax.ShapeDtypeStruct((N, D), jnp.float32),
    )
    gate = pl.pallas_call(
        _gate_body,
        grid=grid,
        in_specs=[row_spec, w_spec, b_spec],
        out_specs=row_spec,
        out_shape=jax.ShapeDtypeStruct((N, D), jnp.float32),
    )

    xl1 = mm(x, W1)
    g1 = gate(x, Wres1, bres1r)

    xs0, xs1 = pl.pallas_call(
        _tc1b_body,
        grid=grid,
        in_specs=[row_spec, cnt_spec, cnt_spec],
        out_specs=[half_spec, half_spec],
        out_shape=[jax.ShapeDtypeStruct((N, HALF), jnp.float32),
                   jax.ShapeDtypeStruct((N, HALF), jnp.float32)],
    )(xl1, c0, c1)

    scatter = _make_scatter(N, E, HALF)
    S0, S1 = scatter(row, col, xs0, xs1, zeros_half)

    h1, ys0, ys1 = pl.pallas_call(
        _tc2a_body,
        grid=grid,
        in_specs=[row_spec, row_spec, half_spec, half_spec, half_spec,
                  half_spec, b_spec, cnt_spec, cnt_spec, w_spec],
        out_specs=[row_spec, half_spec, half_spec],
        out_shape=[jax.ShapeDtypeStruct((N, D), jnp.float32),
                   jax.ShapeDtypeStruct((N, HALF), jnp.float32),
                   jax.ShapeDtypeStruct((N, HALF), jnp.float32)],
    )(x, g1, S0, S1, xs0, xs1, b1r, c0, c1, W2)

    g2 = gate(h1, Wres2, bres2r)
    T0, T1 = scatter(row, col, ys0, ys1, zeros_half)

    out = pl.pallas_call(
        _tc3_body,
        grid=grid,
        in_specs=[row_spec, row_spec, half_spec, half_spec, half_spec,
                  half_spec, b_spec, cnt_spec, cnt_spec],
        out_specs=row_spec,
        out_shape=jax.ShapeDtypeStruct((N, D), jnp.float32),
    )(h1, g2, T0, T1, ys0, ys1, b2r, c0, c1)

    return out
